# Initial kernel scaffold; baseline (speedup 1.0000x reference)
import jax
import jax.numpy as jnp
from jax.experimental import pallas as pl


def kernel(x, formal_charge, segment_ids):
    raise NotImplementedError("write your pallas kernel here")


# speedup vs baseline: 7.1054x; 7.1054x over previous
import functools

import jax
import jax.numpy as jnp
from jax import lax
from jax.experimental import pallas as pl
from jax.experimental.pallas import tpu as pltpu
from jax.experimental.pallas import tpu_sc as plsc

N = 1600000
SEG = 50000
NC, NS, L = 2, 16, 16
NW = NC * NS
CHUNK = N // NW
BLK = 10000
NBLK = CHUNK // BLK
GRP = BLK // L
SLICE = 3136
PAD_SEG = NS * SLICE

_mesh = plsc.VectorSubcoreMesh(core_axis_name="c", subcore_axis_name="s")
_params = pltpu.CompilerParams(needs_layout_passes=False)


@functools.partial(
    pl.kernel,
    out_type=(
        jax.ShapeDtypeStruct((NC * PAD_SEG,), jnp.float32),
        jax.ShapeDtypeStruct((NC * PAD_SEG,), jnp.float32),
    ),
    mesh=_mesh,
    compiler_params=_params,
    scratch_types=[
        pltpu.VMEM((2 * BLK,), jnp.float32),
        pltpu.VMEM((BLK,), jnp.int32),
        pltpu.VMEM((BLK,), jnp.int32),
        pltpu.VMEM((BLK,), jnp.float32),
        pltpu.VMEM((BLK,), jnp.float32),
        pltpu.VMEM((SLICE,), jnp.float32),
        pltpu.VMEM_SHARED((PAD_SEG,), jnp.float32),
        pltpu.VMEM_SHARED((PAD_SEG,), jnp.float32),
    ],
)
def _phase1(x_hbm, fc_hbm, sid_hbm, outA, outB, xfv, fcv, sidv, valv, invv,
            zbuf, accA, accB):
    c = lax.axis_index("c")
    s = lax.axis_index("s")
    wid = c * NS + s
    iota = lax.iota(jnp.int32, 16)

    def _zfill(j, _):
        zbuf[pl.ds(j * 16, 16)] = jnp.zeros((16,), jnp.float32)
        return 0
    lax.fori_loop(0, SLICE // 16, _zfill, 0)
    pltpu.sync_copy(zbuf, accA.at[pl.ds(s * SLICE, SLICE)])
    pltpu.sync_copy(zbuf, accB.at[pl.ds(s * SLICE, SLICE)])
    plsc.subcore_barrier()

    for blk in range(NBLK):
        st = wid * CHUNK + blk * BLK
        pltpu.sync_copy(x_hbm.at[pl.ds(2 * st, 2 * BLK)], xfv)
        pltpu.sync_copy(fc_hbm.at[pl.ds(st, BLK)], fcv)
        pltpu.sync_copy(sid_hbm.at[pl.ds(st, BLK)], sidv)

        def _grp(j, _):
            d = pl.ds(j * 16, 16)
            idx2 = (j * 32) + 2 * iota
            e = plsc.load_gather(xfv, [idx2])
            h = plsc.load_gather(xfv, [idx2 + 1])
            inv = 1.0 / h
            fcf = fcv[d].astype(jnp.float32)
            valv[d] = inv * e + fcf
            invv[d] = inv
            return 0
        lax.fori_loop(0, GRP, _grp, 0)

        pltpu.sync_copy(valv, accA.at[sidv], add=True)
        pltpu.sync_copy(invv, accB.at[sidv], add=True)

    plsc.subcore_barrier()
    pltpu.sync_copy(accA.at[pl.ds(s * SLICE, SLICE)], zbuf)
    pltpu.sync_copy(zbuf, outA.at[pl.ds(c * PAD_SEG + s * SLICE, SLICE)])
    pltpu.sync_copy(accB.at[pl.ds(s * SLICE, SLICE)], zbuf)
    pltpu.sync_copy(zbuf, outB.at[pl.ds(c * PAD_SEG + s * SLICE, SLICE)])


@functools.partial(
    pl.kernel,
    out_type=jax.ShapeDtypeStruct((N,), jnp.float32),
    mesh=_mesh,
    compiler_params=_params,
    scratch_types=[
        pltpu.VMEM((SLICE,), jnp.float32),
        pltpu.VMEM((SLICE,), jnp.float32),
        pltpu.VMEM((SLICE,), jnp.float32),
        pltpu.VMEM((SLICE,), jnp.float32),
        pltpu.VMEM((SLICE,), jnp.float32),
        pltpu.VMEM((PAD_SEG,), jnp.float32),
        pltpu.VMEM((2 * BLK,), jnp.float32),
        pltpu.VMEM((BLK,), jnp.int32),
        pltpu.VMEM((BLK,), jnp.float32),
        pltpu.VMEM_SHARED((PAD_SEG,), jnp.float32),
    ],
)
def _phase2(x_hbm, sid_hbm, pA, pB, out_hbm, a0, a1, b0, b1, pmv, pmfull,
            xfv, sidv, outv, pm_sh):
    c = lax.axis_index("c")
    s = lax.axis_index("s")
    wid = c * NS + s
    iota = lax.iota(jnp.int32, 16)

    pltpu.sync_copy(pA.at[pl.ds(s * SLICE, SLICE)], a0)
    pltpu.sync_copy(pA.at[pl.ds(PAD_SEG + s * SLICE, SLICE)], a1)
    pltpu.sync_copy(pB.at[pl.ds(s * SLICE, SLICE)], b0)
    pltpu.sync_copy(pB.at[pl.ds(PAD_SEG + s * SLICE, SLICE)], b1)

    def _pm(j, _):
        d = pl.ds(j * 16, 16)
        pmv[d] = (a0[d] + a1[d]) / (b0[d] + b1[d])
        return 0
    lax.fori_loop(0, SLICE // 16, _pm, 0)
    pltpu.sync_copy(pmv, pm_sh.at[pl.ds(s * SLICE, SLICE)])
    plsc.subcore_barrier()

    pltpu.sync_copy(pm_sh, pmfull)

    for blk in range(NBLK):
        st = wid * CHUNK + blk * BLK
        pltpu.sync_copy(x_hbm.at[pl.ds(2 * st, 2 * BLK)], xfv)
        pltpu.sync_copy(sid_hbm.at[pl.ds(st, BLK)], sidv)

        def _grp(j, _):
            d = pl.ds(j * 16, 16)
            idx2 = (j * 32) + 2 * iota
            pmg = plsc.load_gather(pmfull, [sidv[d]])
            e = plsc.load_gather(xfv, [idx2])
            h = plsc.load_gather(xfv, [idx2 + 1])
            inv = 1.0 / h
            outv[d] = inv * (pmg - e)
            return 0
        lax.fori_loop(0, GRP, _grp, 0)
        pltpu.sync_copy(outv, out_hbm.at[pl.ds(st, BLK)])


@jax.jit
def kernel(x, formal_charge, segment_ids):
    sid = segment_ids.astype(jnp.int32)
    fc = formal_charge.astype(jnp.int32)
    xf = x.reshape(-1)
    pA, pB = _phase1(xf, fc, sid)
    charges = _phase2(xf, sid, pA, pB)
    return charges.reshape(-1, 1)


# speedup vs baseline: 63.9404x; 8.9988x over previous
import functools

import jax
import jax.numpy as jnp
from jax import lax
from jax.experimental import pallas as pl
from jax.experimental.pallas import tpu as pltpu
from jax.experimental.pallas import tpu_sc as plsc

N = 1600000
SEG = 50000
NC, NS, L = 2, 16, 16
NW = NC * NS
CHUNK = N // NW
BLK = 10000
NBLK = CHUNK // BLK
GRP = BLK // L
SLICE = 3136
PAD_SEG = NS * SLICE

_mesh = plsc.VectorSubcoreMesh(core_axis_name="c", subcore_axis_name="s")
_params = pltpu.CompilerParams(needs_layout_passes=False)


@functools.partial(
    pl.kernel,
    out_type=(
        jax.ShapeDtypeStruct((NC * PAD_SEG,), jnp.float32),
        jax.ShapeDtypeStruct((NC * PAD_SEG,), jnp.float32),
    ),
    mesh=_mesh,
    compiler_params=_params,
    scratch_types=[
        pltpu.VMEM((BLK,), jnp.float32),
        pltpu.VMEM((BLK,), jnp.float32),
        pltpu.VMEM((BLK,), jnp.int32),
        pltpu.VMEM((BLK,), jnp.int32),
        pltpu.VMEM((BLK,), jnp.float32),
        pltpu.VMEM((BLK,), jnp.float32),
        pltpu.VMEM((SLICE,), jnp.float32),
        pltpu.VMEM_SHARED((PAD_SEG,), jnp.float32),
        pltpu.VMEM_SHARED((PAD_SEG,), jnp.float32),
    ],
)
def _phase1(e_hbm, h_hbm, fc_hbm, sid_hbm, outA, outB, ev, hv, fcv, sidv, valv, invv,
            zbuf, accA, accB):
    c = lax.axis_index("c")
    s = lax.axis_index("s")
    wid = c * NS + s

    def _zfill(j, _):
        zbuf[pl.ds(j * 16, 16)] = jnp.zeros((16,), jnp.float32)
        return 0
    lax.fori_loop(0, SLICE // 16, _zfill, 0)
    pltpu.sync_copy(zbuf, accA.at[pl.ds(s * SLICE, SLICE)])
    pltpu.sync_copy(zbuf, accB.at[pl.ds(s * SLICE, SLICE)])
    plsc.subcore_barrier()

    for blk in range(NBLK):
        st = wid * CHUNK + blk * BLK
        pltpu.sync_copy(e_hbm.at[pl.ds(st, BLK)], ev)
        pltpu.sync_copy(h_hbm.at[pl.ds(st, BLK)], hv)
        pltpu.sync_copy(fc_hbm.at[pl.ds(st, BLK)], fcv)
        pltpu.sync_copy(sid_hbm.at[pl.ds(st, BLK)], sidv)

        def _grp(j, _):
            d = pl.ds(j * 16, 16)
            e = ev[d]
            inv = 1.0 / hv[d]
            fcf = fcv[d].astype(jnp.float32)
            valv[d] = inv * e + fcf
            invv[d] = inv
            return 0
        lax.fori_loop(0, GRP, _grp, 0)

        pltpu.sync_copy(valv, accA.at[sidv], add=True)
        pltpu.sync_copy(invv, accB.at[sidv], add=True)

    plsc.subcore_barrier()
    pltpu.sync_copy(accA.at[pl.ds(s * SLICE, SLICE)], zbuf)
    pltpu.sync_copy(zbuf, outA.at[pl.ds(c * PAD_SEG + s * SLICE, SLICE)])
    pltpu.sync_copy(accB.at[pl.ds(s * SLICE, SLICE)], zbuf)
    pltpu.sync_copy(zbuf, outB.at[pl.ds(c * PAD_SEG + s * SLICE, SLICE)])


@functools.partial(
    pl.kernel,
    out_type=jax.ShapeDtypeStruct((N,), jnp.float32),
    mesh=_mesh,
    compiler_params=_params,
    scratch_types=[
        pltpu.VMEM((SLICE,), jnp.float32),
        pltpu.VMEM((SLICE,), jnp.float32),
        pltpu.VMEM((SLICE,), jnp.float32),
        pltpu.VMEM((SLICE,), jnp.float32),
        pltpu.VMEM((SLICE,), jnp.float32),
        pltpu.VMEM((PAD_SEG,), jnp.float32),
        pltpu.VMEM((BLK,), jnp.float32),
        pltpu.VMEM((BLK,), jnp.float32),
        pltpu.VMEM((BLK,), jnp.int32),
        pltpu.VMEM((BLK,), jnp.float32),
        pltpu.VMEM_SHARED((PAD_SEG,), jnp.float32),
    ],
)
def _phase2(e_hbm, h_hbm, sid_hbm, pA, pB, out_hbm, a0, a1, b0, b1, pmv, pmfull,
            ev, hv, sidv, outv, pm_sh):
    c = lax.axis_index("c")
    s = lax.axis_index("s")
    wid = c * NS + s

    pltpu.sync_copy(pA.at[pl.ds(s * SLICE, SLICE)], a0)
    pltpu.sync_copy(pA.at[pl.ds(PAD_SEG + s * SLICE, SLICE)], a1)
    pltpu.sync_copy(pB.at[pl.ds(s * SLICE, SLICE)], b0)
    pltpu.sync_copy(pB.at[pl.ds(PAD_SEG + s * SLICE, SLICE)], b1)

    def _pm(j, _):
        d = pl.ds(j * 16, 16)
        pmv[d] = (a0[d] + a1[d]) / (b0[d] + b1[d])
        return 0
    lax.fori_loop(0, SLICE // 16, _pm, 0)
    pltpu.sync_copy(pmv, pm_sh.at[pl.ds(s * SLICE, SLICE)])
    plsc.subcore_barrier()

    pltpu.sync_copy(pm_sh, pmfull)

    for blk in range(NBLK):
        st = wid * CHUNK + blk * BLK
        pltpu.sync_copy(e_hbm.at[pl.ds(st, BLK)], ev)
        pltpu.sync_copy(h_hbm.at[pl.ds(st, BLK)], hv)
        pltpu.sync_copy(sid_hbm.at[pl.ds(st, BLK)], sidv)

        def _grp(j, _):
            d = pl.ds(j * 16, 16)
            pmg = plsc.load_gather(pmfull, [sidv[d]])
            inv = 1.0 / hv[d]
            outv[d] = inv * (pmg - ev[d])
            return 0
        lax.fori_loop(0, GRP, _grp, 0)
        pltpu.sync_copy(outv, out_hbm.at[pl.ds(st, BLK)])


@jax.jit
def kernel(x, formal_charge, segment_ids):
    sid = segment_ids.astype(jnp.int32)
    fc = formal_charge.astype(jnp.int32)
    e = x[:, 0]
    h = x[:, 1]
    pA, pB = _phase1(e, h, fc, sid)
    charges = _phase2(e, h, sid, pA, pB)
    return charges.reshape(-1, 1)


# speedup vs baseline: 64.6823x; 1.0116x over previous
import functools

import jax
import jax.numpy as jnp
from jax import lax
from jax.experimental import pallas as pl
from jax.experimental.pallas import tpu as pltpu
from jax.experimental.pallas import tpu_sc as plsc

N = 1600000
SEG = 50000
NC, NS, L = 2, 16, 16
NW = NC * NS
CHUNK = N // NW
BLK = 10000
NBLK = CHUNK // BLK
GRP = BLK // L
SLICE = 3136
PAD_SEG = NS * SLICE
HALF = N // 2
FB = 2048
FGRP = FB // L

_mesh = plsc.VectorSubcoreMesh(core_axis_name="c", subcore_axis_name="s")
_params = pltpu.CompilerParams(needs_layout_passes=False)


@functools.partial(
    pl.kernel,
    out_type=jax.ShapeDtypeStruct((N,), jnp.float32),
    mesh=_mesh,
    compiler_params=_params,
    scratch_types=[
        pltpu.VMEM((BLK,), jnp.float32),
        pltpu.VMEM((BLK,), jnp.float32),
        pltpu.VMEM((BLK,), jnp.int32),
        pltpu.VMEM((BLK,), jnp.int32),
        pltpu.VMEM((BLK,), jnp.float32),
        pltpu.VMEM((BLK,), jnp.float32),
        pltpu.VMEM((SLICE,), jnp.float32),
        pltpu.VMEM((16,), jnp.int32),
        pltpu.VMEM((16,), jnp.float32),
        pltpu.VMEM((16,), jnp.float32),
        pltpu.VMEM((PAD_SEG,), jnp.float32),
        pltpu.VMEM_SHARED((PAD_SEG,), jnp.float32),
        pltpu.VMEM_SHARED((PAD_SEG,), jnp.float32),
        pltpu.VMEM_SHARED((PAD_SEG,), jnp.float32),
    ],
)
def _fused(e_hbm, h_hbm, fc_hbm, sid_hbm, out_hbm, ev, hv, fcv, sidv, valv,
           invv, zbuf, fixidx, fixA, fixB, pmfull, accA, accB, pm_sh):
    c = lax.axis_index("c")
    s = lax.axis_index("s")
    wid = c * NS + s

    def _zfill(j, _):
        zbuf[pl.ds(j * 16, 16)] = jnp.zeros((16,), jnp.float32)
        return 0
    lax.fori_loop(0, SLICE // 16, _zfill, 0)
    pltpu.sync_copy(zbuf, accA.at[pl.ds(s * SLICE, SLICE)])
    pltpu.sync_copy(zbuf, accB.at[pl.ds(s * SLICE, SLICE)])
    plsc.subcore_barrier()

    for blk in range(NBLK):
        st = wid * CHUNK + blk * BLK
        pltpu.sync_copy(e_hbm.at[pl.ds(st, BLK)], ev)
        pltpu.sync_copy(h_hbm.at[pl.ds(st, BLK)], hv)
        pltpu.sync_copy(fc_hbm.at[pl.ds(st, BLK)], fcv)
        pltpu.sync_copy(sid_hbm.at[pl.ds(st, BLK)], sidv)

        def _grp(j, _):
            d = pl.ds(j * 16, 16)
            inv = 1.0 / hv[d]
            valv[d] = inv * ev[d] + fcv[d].astype(jnp.float32)
            invv[d] = inv
            return 0
        lax.fori_loop(0, GRP, _grp, 0)

        pltpu.sync_copy(valv, accA.at[sidv], add=True)
        pltpu.sync_copy(invv, accB.at[sidv], add=True)

    plsc.subcore_barrier()

    @pl.when(s == 0)
    def _fixup():
        pltpu.sync_copy(sid_hbm.at[pl.ds(HALF - 8, 16)], fixidx)
        bv = fixidx[pl.ds(0, 16)]
        sid_l = bv[7]
        sid_r = bv[8]

        @pl.when(sid_l == sid_r)
        def _straddle():
            sv = jnp.full((16,), sid_l, jnp.int32)
            fwd = c == 0

            def _cond(carry):
                t, go, _, _ = carry
                return go & (t < HALF // FB)

            def _body(carry):
                t, go, vA, vB = carry
                off = jnp.where(fwd, HALF + t * FB, HALF - (t + 1) * FB)
                pltpu.sync_copy(e_hbm.at[pl.ds(off, FB)], ev.at[pl.ds(0, FB)])
                pltpu.sync_copy(h_hbm.at[pl.ds(off, FB)], hv.at[pl.ds(0, FB)])
                pltpu.sync_copy(fc_hbm.at[pl.ds(off, FB)], fcv.at[pl.ds(0, FB)])
                pltpu.sync_copy(sid_hbm.at[pl.ds(off, FB)], sidv.at[pl.ds(0, FB)])

                def _fgrp(j, fcarry):
                    fvA, fvB, nmatch = fcarry
                    d = pl.ds(j * 16, 16)
                    m = sidv[d] == sv
                    inv = 1.0 / hv[d]
                    val = inv * ev[d] + fcv[d].astype(jnp.float32)
                    zf = jnp.zeros((16,), jnp.float32)
                    fvA = fvA + jnp.where(m, val, zf)
                    fvB = fvB + jnp.where(m, inv, zf)
                    nmatch = nmatch + jnp.sum(m.astype(jnp.int32))
                    return fvA, fvB, nmatch

                vA, vB, nmatch = lax.fori_loop(
                    0, FGRP, _fgrp, (vA, vB, jnp.int32(0)))
                return t + 1, go & (nmatch == FB), vA, vB

            zf16 = jnp.zeros((16,), jnp.float32)
            _, _, vA, vB = lax.while_loop(
                _cond, _body, (jnp.int32(0), jnp.bool_(True), zf16, zf16))

            lane = lax.iota(jnp.int32, 16)
            first = (lane == 0).astype(jnp.float32)
            fixidx[:] = sv
            fixA[:] = jnp.sum(vA) * first
            fixB[:] = jnp.sum(vB) * first
            pltpu.sync_copy(fixA, accA.at[fixidx], add=True)
            pltpu.sync_copy(fixB, accB.at[fixidx], add=True)

    plsc.subcore_barrier()

    sl = pl.ds(s * SLICE, SLICE)
    pltpu.sync_copy(accA.at[sl], valv.at[pl.ds(0, SLICE)])
    pltpu.sync_copy(accB.at[sl], invv.at[pl.ds(0, SLICE)])

    def _pm(j, _):
        d = pl.ds(j * 16, 16)
        zbuf[d] = valv[d] / invv[d]
        return 0
    lax.fori_loop(0, SLICE // 16, _pm, 0)
    pltpu.sync_copy(zbuf, pm_sh.at[sl])
    plsc.subcore_barrier()

    pltpu.sync_copy(pm_sh, pmfull)

    for blk in range(NBLK):
        st = wid * CHUNK + blk * BLK
        pltpu.sync_copy(e_hbm.at[pl.ds(st, BLK)], ev)
        pltpu.sync_copy(h_hbm.at[pl.ds(st, BLK)], hv)
        pltpu.sync_copy(sid_hbm.at[pl.ds(st, BLK)], sidv)

        def _out(j, _):
            d = pl.ds(j * 16, 16)
            pmg = plsc.load_gather(pmfull, [sidv[d]])
            inv = 1.0 / hv[d]
            valv[d] = inv * (pmg - ev[d])
            return 0
        lax.fori_loop(0, GRP, _out, 0)
        pltpu.sync_copy(valv, out_hbm.at[pl.ds(st, BLK)])


@jax.jit
def kernel(x, formal_charge, segment_ids):
    sid = segment_ids.astype(jnp.int32)
    fc = formal_charge.astype(jnp.int32)
    e = x[:, 0]
    h = x[:, 1]
    charges = _fused(e, h, fc, sid)
    return charges.reshape(-1, 1)
